# Initial kernel scaffold; baseline (speedup 1.0000x reference)
import dataclasses
import functools

import jax
import jax.numpy as jnp
from jax.experimental import pallas as pl
from jax.experimental.pallas import tpu as pltpu
from jax.experimental.pallas import tpu_sc as plsc

_NUM_POINTS = (4096, 1024, 256, 64)
_RADII = ((0.1, 0.5), (0.5, 1.0), (1.0, 2.0), (2.0, 4.0))
_NSAMPLES = ((16, 32), (16, 32), (16, 32), (16, 32))
_BN_EPS = 1e-3
_BIG = 3.0e7


def _pow2_floor(v):
    p = 1
    while p * 2 <= v:
        p *= 2
    return p


def _sc_gather(table, idx):
    num = idx.shape[0]
    c = table.shape[1]
    window = min(128, max(16, _pow2_floor(16384 // c)))
    chunk = window * 32
    nump = ((num + chunk - 1) // chunk) * chunk
    if nump != num:
        idx = jnp.concatenate([idx, jnp.zeros((nump - num,), jnp.int32)])
    idx2 = idx.reshape(1, nump)

    mesh = plsc.VectorSubcoreMesh(core_axis_name="core", subcore_axis_name="subcore")
    cp = pltpu.CompilerParams()
    if "needs_layout_passes" in pltpu.CompilerParams.__dataclass_fields__:
        cp = dataclasses.replace(cp, needs_layout_passes=False)

    @functools.partial(
        pl.kernel,
        out_type=jax.ShapeDtypeStruct((nump, c), table.dtype),
        mesh=mesh,
        compiler_params=cp,
    )
    def gather_kernel(x_hbm, i_hbm, o_hbm):
        def body(i_vmem, o_vmem):
            pltpu.sync_copy(x_hbm.at[i_vmem.at[0]], o_vmem)

        pltpu.emit_pipeline(
            body,
            grid=(nump // window,),
            in_specs=[pl.BlockSpec((1, window), index_map=lambda i: (0, i))],
            out_specs=[pl.BlockSpec((window, c), index_map=lambda i: (i, 0))],
            core_axis_name=("core", "subcore"),
            dimension_semantics=(pltpu.PARALLEL,),
        )(i_hbm, o_hbm)

    out = gather_kernel(table, idx2)
    return out[:num] if nump != num else out


def _fps(xs, ys, zs, npoint):
    b, r, cols = xs.shape
    n = r * cols
    rm = max(1, npoint // 128)
    cm = min(npoint, 128)

    def body(x_ref, y_ref, z_ref, o_ref, dist_ref):
        bb = pl.program_id(0)
        base = bb * n
        x = x_ref[0]
        y = y_ref[0]
        z = z_ref[0]
        dist_ref[...] = jnp.full((r, cols), 1e10, jnp.float32)
        gr = jax.lax.broadcasted_iota(jnp.float32, (r, cols), 0)
        gc = jax.lax.broadcasted_iota(jnp.float32, (r, cols), 1)
        gidx = gr * float(cols) + gc
        oi = jax.lax.broadcasted_iota(jnp.int32, (rm, cm), 0) * cm + \
            jax.lax.broadcasted_iota(jnp.int32, (rm, cm), 1)
        o_ref[0] = jnp.full((rm, cm), base, jnp.int32)

        sel0 = gidx == 0.0
        c0 = (
            jnp.sum(jnp.where(sel0, x, 0.0)),
            jnp.sum(jnp.where(sel0, y, 0.0)),
            jnp.sum(jnp.where(sel0, z, 0.0)),
        )

        def step(i, carry):
            cx, cy, cz = carry
            dx = x - cx
            dy = y - cy
            dz = z - cz
            d = dx * dx + dy * dy + dz * dz
            nd = jnp.minimum(dist_ref[...], d)
            dist_ref[...] = nd
            mx = jnp.max(nd)
            nxtf = jnp.min(jnp.where(nd == mx, gidx, _BIG))
            nxt = nxtf.astype(jnp.int32)
            o_ref[0] = jnp.where(oi == i, nxt + base, o_ref[0])
            sel = gidx == nxtf
            return (
                jnp.sum(jnp.where(sel, x, 0.0)),
                jnp.sum(jnp.where(sel, y, 0.0)),
                jnp.sum(jnp.where(sel, z, 0.0)),
            )

        jax.lax.fori_loop(1, npoint, step, c0)

    out = pl.pallas_call(
        body,
        grid=(b,),
        in_specs=[pl.BlockSpec((1, r, cols), lambda i: (i, 0, 0))] * 3,
        out_specs=pl.BlockSpec((1, rm, cm), lambda i: (i, 0, 0)),
        out_shape=jax.ShapeDtypeStruct((b, rm, cm), jnp.int32),
        scratch_shapes=[pltpu.VMEM((r, cols), jnp.float32)],
    )(xs, ys, zs)
    return out.reshape(b * npoint)


def _sa_select(px, py, pz, qtab, n, m, r1, r2, k1, k2, m_blk):
    b = px.shape[0]
    cq = qtab.shape[1]
    nf = float(n)
    nb = m // m_blk

    def body(px_ref, py_ref, pz_ref, q_ref, o1_ref, o2_ref, s1_ref, s2_ref):
        bb = pl.program_id(0)
        base = bb * n
        qx = q_ref[:, 0:1]
        qy = q_ref[:, 1:2]
        qz = q_ref[:, 2:3]
        qn = (qx * qx + qy * qy) + qz * qz
        x = px_ref[0]
        y = py_ref[0]
        z = pz_ref[0]
        pn = (x * x + y * y) + z * z
        ab = (qx * x + qy * y) + qz * z
        d2 = (qn + pn) - 2.0 * ab
        iota = jax.lax.broadcasted_iota(jnp.float32, (m_blk, n), 1)
        s1_ref[...] = jnp.where(d2 < r1 * r1, iota, nf)
        s2_ref[...] = jnp.where(d2 < r2 * r2, iota, nf)

        def extract(s_ref, k, o_ref):
            lane = jax.lax.broadcasted_iota(jnp.int32, (m_blk, k), 1)

            def cond(c):
                s, alive, _ = c
                return jnp.logical_and(s < k, alive)

            def bod(c):
                s, _, out = c
                mv = jnp.min(s_ref[...], axis=1, keepdims=True)
                found = mv < nf
                s_ref[...] = jnp.where(s_ref[...] == mv, _BIG, s_ref[...])
                out = jnp.where(
                    jnp.logical_and(lane == s, found), mv.astype(jnp.int32), out
                )
                return (s + 1, jnp.any(found), out)

            out0 = jnp.full((m_blk, k), -1, jnp.int32)
            _, _, out = jax.lax.while_loop(cond, bod, (0, True, out0))
            first = jnp.maximum(out[:, 0:1], 0)
            out = jnp.where(out >= 0, out, first)
            o_ref[...] = out + base

        extract(s1_ref, k1, o1_ref)
        extract(s2_ref, k2, o2_ref)

    o1, o2 = pl.pallas_call(
        body,
        grid=(b, nb),
        in_specs=[
            pl.BlockSpec((1, 1, n), lambda i, j: (i, 0, 0)),
            pl.BlockSpec((1, 1, n), lambda i, j: (i, 0, 0)),
            pl.BlockSpec((1, 1, n), lambda i, j: (i, 0, 0)),
            pl.BlockSpec((m_blk, cq), lambda i, j: (i * nb + j, 0)),
        ],
        out_specs=[
            pl.BlockSpec((m_blk, k1), lambda i, j: (i * nb + j, 0)),
            pl.BlockSpec((m_blk, k2), lambda i, j: (i * nb + j, 0)),
        ],
        out_shape=[
            jax.ShapeDtypeStruct((b * m, k1), jnp.int32),
            jax.ShapeDtypeStruct((b * m, k2), jnp.int32),
        ],
        scratch_shapes=[
            pltpu.VMEM((m_blk, n), jnp.float32),
            pltpu.VMEM((m_blk, n), jnp.float32),
        ],
    )(px, py, pz, qtab)
    return o1, o2


def _sa_mlp(g, qtab, w1, w1q, b1, w2, b2, w3, b3, k, m_blk):
    rows = g.shape[0] // k
    cq = qtab.shape[1]
    cp = g.shape[1]
    c1 = w1.shape[1]
    c2 = w2.shape[1]
    c3 = w3.shape[1]
    nb = rows // m_blk

    def body(g_ref, q_ref, w1_ref, w1q_ref, b1_ref, w2_ref, b2_ref, w3_ref,
             b3_ref, o_ref):
        gv = g_ref[...]
        a = jnp.dot(gv, w1_ref[...], preferred_element_type=jnp.float32)
        bq = jnp.dot(q_ref[...], w1q_ref[...], preferred_element_type=jnp.float32)
        h = a.reshape(m_blk, k, c1) - bq[:, None, :] + b1_ref[...][None]
        h = jnp.maximum(h, 0.0).reshape(m_blk * k, c1)
        h = jnp.dot(h, w2_ref[...], preferred_element_type=jnp.float32) + b2_ref[...]
        h = jnp.maximum(h, 0.0)
        h = jnp.dot(h, w3_ref[...], preferred_element_type=jnp.float32) + b3_ref[...]
        h = jnp.maximum(h, 0.0)
        o_ref[...] = jnp.max(h.reshape(m_blk, k, c3), axis=1)

    return pl.pallas_call(
        body,
        grid=(nb,),
        in_specs=[
            pl.BlockSpec((m_blk * k, cp), lambda i: (i, 0)),
            pl.BlockSpec((m_blk, cq), lambda i: (i, 0)),
            pl.BlockSpec((cp, c1), lambda i: (0, 0)),
            pl.BlockSpec((cq, c1), lambda i: (0, 0)),
            pl.BlockSpec((1, c1), lambda i: (0, 0)),
            pl.BlockSpec((c1, c2), lambda i: (0, 0)),
            pl.BlockSpec((1, c2), lambda i: (0, 0)),
            pl.BlockSpec((c2, c3), lambda i: (0, 0)),
            pl.BlockSpec((1, c3), lambda i: (0, 0)),
        ],
        out_specs=pl.BlockSpec((m_blk, c3), lambda i: (i, 0)),
        out_shape=jax.ShapeDtypeStruct((rows, c3), jnp.float32),
    )(g, qtab, w1, w1q, b1, w2, b2, w3, b3)


def _fp_select(sx, sy, sz, ttab, s_pts, t_pts, t_blk):
    b = sx.shape[0]
    cq = ttab.shape[1]
    nb = t_pts // t_blk

    def body(sx_ref, sy_ref, sz_ref, t_ref, oi_ref, od_ref, d2_ref):
        bb = pl.program_id(0)
        base = bb * s_pts
        qx = t_ref[:, 0:1]
        qy = t_ref[:, 1:2]
        qz = t_ref[:, 2:3]
        qn = (qx * qx + qy * qy) + qz * qz
        x = sx_ref[0]
        y = sy_ref[0]
        z = sz_ref[0]
        pn = (x * x + y * y) + z * z
        ab = (qx * x + qy * y) + qz * z
        d2_ref[...] = (qn + pn) - 2.0 * ab
        iota = jax.lax.broadcasted_iota(jnp.float32, (t_blk, s_pts), 1)
        lane = jax.lax.broadcasted_iota(jnp.int32, (t_blk, 8), 1)
        oi = jnp.zeros((t_blk, 8), jnp.int32)
        od = jnp.zeros((t_blk, 8), jnp.float32)
        for s in range(3):
            mv = jnp.min(d2_ref[...], axis=1, keepdims=True)
            idxf = jnp.min(
                jnp.where(d2_ref[...] == mv, iota, _BIG), axis=1, keepdims=True
            )
            d2_ref[...] = jnp.where(iota == idxf, _BIG, d2_ref[...])
            oi = jnp.where(lane == s, idxf.astype(jnp.int32) + base, oi)
            od = jnp.where(lane == s, jnp.maximum(mv, 0.0), od)
        oi_ref[...] = oi
        od_ref[...] = od

    oi, od = pl.pallas_call(
        body,
        grid=(b, nb),
        in_specs=[
            pl.BlockSpec((1, 1, s_pts), lambda i, j: (i, 0, 0)),
            pl.BlockSpec((1, 1, s_pts), lambda i, j: (i, 0, 0)),
            pl.BlockSpec((1, 1, s_pts), lambda i, j: (i, 0, 0)),
            pl.BlockSpec((t_blk, cq), lambda i, j: (i * nb + j, 0)),
        ],
        out_specs=[
            pl.BlockSpec((t_blk, 8), lambda i, j: (i * nb + j, 0)),
            pl.BlockSpec((t_blk, 8), lambda i, j: (i * nb + j, 0)),
        ],
        out_shape=[
            jax.ShapeDtypeStruct((b * t_pts, 8), jnp.int32),
            jax.ShapeDtypeStruct((b * t_pts, 8), jnp.float32),
        ],
        scratch_shapes=[pltpu.VMEM((t_blk, s_pts), jnp.float32)],
    )(sx, sy, sz, ttab)
    return oi, od


def _fp_mlp(sf, dd, skip, w1a, w1b, b1, w2, b2, m_blk):
    rows = dd.shape[0]
    cs = sf.shape[1]
    csk = skip.shape[1]
    c1 = w1a.shape[1]
    c2 = w2.shape[1]
    nb = rows // m_blk

    def body(sf_ref, d_ref, sk_ref, w1a_ref, w1b_ref, b1_ref, w2_ref, b2_ref,
             o_ref):
        d = d_ref[...][:, 0:3]
        w = 1.0 / (d + 1e-8)
        ws = (w[:, 0:1] + w[:, 1:2]) + w[:, 2:3]
        w = w / ws
        sf3 = sf_ref[...].reshape(m_blk, 3, cs)
        interp = (sf3[:, 0, :] * w[:, 0:1] + sf3[:, 1, :] * w[:, 1:2]) + \
            sf3[:, 2, :] * w[:, 2:3]
        h = jnp.dot(sk_ref[...], w1a_ref[...], preferred_element_type=jnp.float32)
        h = h + jnp.dot(interp, w1b_ref[...], preferred_element_type=jnp.float32)
        h = jnp.maximum(h + b1_ref[...], 0.0)
        h = jnp.dot(h, w2_ref[...], preferred_element_type=jnp.float32) + b2_ref[...]
        o_ref[...] = jnp.maximum(h, 0.0)

    return pl.pallas_call(
        body,
        grid=(nb,),
        in_specs=[
            pl.BlockSpec((m_blk * 3, cs), lambda i: (i, 0)),
            pl.BlockSpec((m_blk, 8), lambda i: (i, 0)),
            pl.BlockSpec((m_blk, csk), lambda i: (i, 0)),
            pl.BlockSpec((csk, c1), lambda i: (0, 0)),
            pl.BlockSpec((cs, c1), lambda i: (0, 0)),
            pl.BlockSpec((1, c1), lambda i: (0, 0)),
            pl.BlockSpec((c1, c2), lambda i: (0, 0)),
            pl.BlockSpec((1, c2), lambda i: (0, 0)),
        ],
        out_specs=pl.BlockSpec((m_blk, c2), lambda i: (i, 0)),
        out_shape=jax.ShapeDtypeStruct((rows, c2), jnp.float32),
    )(sf, dd, skip, w1a, w1b, b1, w2, b2)


def _prep_layer(layer):
    scale = layer["gamma"] / jnp.sqrt(1.0 + _BN_EPS)
    wt = (layer["w"] * scale[:, None]).T
    return wt, layer["beta"][None, :]


def _pad_rows(w, rows):
    return jnp.pad(w, ((0, rows - w.shape[0]), (0, 0)))


def _pad_cols(x, cols):
    return jnp.pad(x, ((0, 0), (0, cols - x.shape[1])))


def _pad16(c):
    return ((c + 15) // 16) * 16


def kernel(points, params):
    b, n0, _ = points.shape
    sizes = (n0,) + _NUM_POINTS

    tbl0 = _pad_cols(points.reshape(b * n0, 5), 16)
    tables = [tbl0]
    feat_tbls = [None]
    coords = []

    def col(tbl, j, npts):
        return tbl[:, j].reshape(b, 1, npts)

    coords.append((col(tbl0, 0, n0), col(tbl0, 1, n0), col(tbl0, 2, n0)))

    for i in range(4):
        n = sizes[i]
        m = sizes[i + 1]
        k1, k2 = _NSAMPLES[i]
        r1, r2 = _RADII[i]
        px, py, pz = coords[i]

        xs = px.reshape(b, n // 128, 128)
        ys = py.reshape(b, n // 128, 128)
        zs = pz.reshape(b, n // 128, 128)
        fps_idx = _fps(xs, ys, zs, m)

        qtab = _sc_gather(tables[i], fps_idx)

        m_blk = min(128, m)
        o1, o2 = _sa_select(px, py, pz, qtab, n, m, r1, r2, k1, k2, m_blk)

        g1 = _sc_gather(tables[i], o1.reshape(-1))
        g2 = _sc_gather(tables[i], o2.reshape(-1))

        outs = []
        for s, (g, kk) in enumerate(((g1, k1), (g2, k2))):
            mlp = params["sa"][i][s]
            w1t, b1 = _prep_layer(mlp[0])
            w2t, b2 = _prep_layer(mlp[1])
            w3t, b3 = _prep_layer(mlp[2])
            cp = tables[i].shape[1]
            cq = qtab.shape[1]
            w1 = _pad_rows(w1t, cp)
            w1q = _pad_rows(w1t[:3], cq)
            mb = max(8, min(128, 2048 // kk, m))
            outs.append(_sa_mlp(g, qtab, w1, w1q, b1, w2t, b2, w3t, b3, kk, mb))
        feat = jnp.concatenate(outs, axis=1)

        tables.append(_pad_cols(jnp.concatenate([qtab[:, :3], feat], axis=1),
                                _pad16(3 + feat.shape[1])))
        feat_tbls.append(feat)
        tq = tables[i + 1]
        coords.append((col(tq, 0, m), col(tq, 1, m), col(tq, 2, m)))

    skip0 = _pad_cols(points[..., 3:].reshape(b * n0, 2), 16)
    skips = [skip0, feat_tbls[1], feat_tbls[2], feat_tbls[3]]
    src_feats = feat_tbls[4]
    fp_outs = []
    for j in range(4):
        tgt = 3 - j
        src = 4 - j
        t_pts = sizes[tgt]
        s_pts = sizes[src]
        sx, sy, sz = coords[src]
        oi, od = _fp_select(sx, sy, sz, tables[tgt], s_pts, t_pts,
                            min(128, t_pts))
        idx3 = oi[:, :3].reshape(-1)
        sf = _sc_gather(src_feats, idx3)
        mlp = params["fp"][j]
        w1t, b1 = _prep_layer(mlp[0])
        w2t, b2 = _prep_layer(mlp[1])
        csk = skips[tgt].shape[1]
        cs = src_feats.shape[1]
        nsk = csk if tgt > 0 else 2
        w1a = _pad_rows(w1t[:nsk], csk)
        w1b = w1t[nsk:]
        assert w1b.shape[0] == cs, (w1b.shape, cs)
        out = _fp_mlp(sf, od, skips[tgt], w1a, w1b, b1, w2t, b2, 128)
        fp_outs.append(out)
        src_feats = out

    xyz_out = tuple(
        tables[i][:, :3].reshape(b, sizes[i], 3) for i in range(4)
    )
    feat_out = tuple(
        fp_outs[3 - i].reshape(b, sizes[i], -1).transpose(0, 2, 1)
        for i in range(4)
    )
    return xyz_out + feat_out


# speedup vs baseline: 17.5447x; 17.5447x over previous
import dataclasses
import functools

import jax
import jax.numpy as jnp
from jax.experimental import pallas as pl
from jax.experimental.pallas import tpu as pltpu
from jax.experimental.pallas import tpu_sc as plsc

_NUM_POINTS = (4096, 1024, 256, 64)
_RADII = ((0.1, 0.5), (0.5, 1.0), (1.0, 2.0), (2.0, 4.0))
_NSAMPLES = ((16, 32), (16, 32), (16, 32), (16, 32))
_BN_EPS = 1e-3
_BIG = 3.0e7


def _pow2_floor(v):
    p = 1
    while p * 2 <= v:
        p *= 2
    return p


def _sc_gather(table, idx):
    num = idx.shape[0]
    c = table.shape[1]
    window = min(128, max(16, _pow2_floor(16384 // c)))
    chunk = window * 32
    nump = ((num + chunk - 1) // chunk) * chunk
    if nump != num:
        idx = jnp.concatenate([idx, jnp.zeros((nump - num,), jnp.int32)])
    idx2 = idx.reshape(1, nump)

    mesh = plsc.VectorSubcoreMesh(core_axis_name="core", subcore_axis_name="subcore")
    cp = pltpu.CompilerParams()
    if "needs_layout_passes" in pltpu.CompilerParams.__dataclass_fields__:
        cp = dataclasses.replace(cp, needs_layout_passes=False)
    if "use_tc_tiling_on_sc" in pltpu.CompilerParams.__dataclass_fields__:
        cp = dataclasses.replace(cp, use_tc_tiling_on_sc=False)

    @functools.partial(
        pl.kernel,
        out_type=jax.ShapeDtypeStruct((nump, c), table.dtype),
        mesh=mesh,
        compiler_params=cp,
    )
    def gather_kernel(x_hbm, i_hbm, o_hbm):
        def body(i_vmem, o_vmem):
            pltpu.sync_copy(x_hbm.at[i_vmem.at[0]], o_vmem)

        pltpu.emit_pipeline(
            body,
            grid=(nump // window,),
            in_specs=[pl.BlockSpec((1, window), index_map=lambda i: (0, i))],
            out_specs=[pl.BlockSpec((window, c), index_map=lambda i: (i, 0))],
            core_axis_name=("core", "subcore"),
            dimension_semantics=(pltpu.PARALLEL,),
        )(i_hbm, o_hbm)

    out = gather_kernel(table, idx2)
    return out[:num] if nump != num else out


def _fps(xs, ys, zs, npoint):
    b, r, cols = xs.shape
    n = r * cols
    rm = max(1, npoint // 128)
    cm = min(npoint, 128)

    def body(x_ref, y_ref, z_ref, o_ref, dist_ref):
        bb = pl.program_id(0)
        base = bb * n
        x = x_ref[0]
        y = y_ref[0]
        z = z_ref[0]
        dist_ref[...] = jnp.full((r, cols), 1e10, jnp.float32)
        gr = jax.lax.broadcasted_iota(jnp.int32, (r, cols), 0).astype(jnp.float32)
        gc = jax.lax.broadcasted_iota(jnp.int32, (r, cols), 1).astype(jnp.float32)
        gidx = gr * float(cols) + gc
        oi = jax.lax.broadcasted_iota(jnp.int32, (rm, cm), 0) * cm + \
            jax.lax.broadcasted_iota(jnp.int32, (rm, cm), 1)
        o_ref[0] = jnp.full((rm, cm), base, jnp.int32)

        sel0 = gidx == 0.0
        c0 = (
            jnp.sum(jnp.where(sel0, x, 0.0)),
            jnp.sum(jnp.where(sel0, y, 0.0)),
            jnp.sum(jnp.where(sel0, z, 0.0)),
        )

        def step(i, carry):
            cx, cy, cz = carry
            dx = x - cx
            dy = y - cy
            dz = z - cz
            d = dx * dx + dy * dy + dz * dz
            nd = jnp.minimum(dist_ref[...], d)
            dist_ref[...] = nd
            mx = jnp.max(nd)
            nxtf = jnp.min(jnp.where(nd == mx, gidx, _BIG))
            nxt = nxtf.astype(jnp.int32)
            o_ref[0] = jnp.where(oi == i, nxt + base, o_ref[0])
            sel = gidx == nxtf
            return (
                jnp.sum(jnp.where(sel, x, 0.0)),
                jnp.sum(jnp.where(sel, y, 0.0)),
                jnp.sum(jnp.where(sel, z, 0.0)),
            )

        jax.lax.fori_loop(1, npoint, step, c0)

    out = pl.pallas_call(
        body,
        grid=(b,),
        in_specs=[pl.BlockSpec((1, r, cols), lambda i: (i, 0, 0))] * 3,
        out_specs=pl.BlockSpec((1, rm, cm), lambda i: (i, 0, 0)),
        out_shape=jax.ShapeDtypeStruct((b, rm, cm), jnp.int32),
        scratch_shapes=[pltpu.VMEM((r, cols), jnp.float32)],
    )(xs, ys, zs)
    return out.reshape(b * npoint)


def _sa_select(px, py, pz, qtab, n, m, r1, r2, k1, k2, m_blk):
    b = px.shape[0]
    cq = qtab.shape[1]
    nf = float(n)
    nb = m // m_blk

    def body(px_ref, py_ref, pz_ref, q_ref, o1_ref, o2_ref, s1_ref, s2_ref):
        bb = pl.program_id(0)
        base = bb * n
        qx = q_ref[:, 0:1]
        qy = q_ref[:, 1:2]
        qz = q_ref[:, 2:3]
        qn = (qx * qx + qy * qy) + qz * qz
        x = px_ref[0]
        y = py_ref[0]
        z = pz_ref[0]
        pn = (x * x + y * y) + z * z
        p3 = jnp.concatenate([x, y, z], axis=0).astype(jnp.bfloat16)
        q3 = q_ref[:, 0:3].astype(jnp.bfloat16)
        ab = jnp.dot(q3, p3, preferred_element_type=jnp.float32)
        d2 = (qn + pn) - 2.0 * ab
        iota = jax.lax.broadcasted_iota(jnp.int32, (m_blk, n), 1).astype(jnp.float32)
        s1_ref[...] = jnp.where(d2 < r1 * r1, iota, nf)
        s2_ref[...] = jnp.where(d2 < r2 * r2, iota, nf)

        def extract(s_ref, k, o_ref):
            lane = jax.lax.broadcasted_iota(jnp.int32, (m_blk, k), 1)

            def cond(c):
                s, alive, _ = c
                return jnp.logical_and(s < k, alive)

            def bod(c):
                s, _, out = c
                mv = jnp.min(s_ref[...], axis=1, keepdims=True)
                found = mv < nf
                s_ref[...] = jnp.where(s_ref[...] == mv, _BIG, s_ref[...])
                out = jnp.where(
                    jnp.logical_and(lane == s, found), mv.astype(jnp.int32), out
                )
                return (s + 1, jnp.any(found), out)

            out0 = jnp.full((m_blk, k), -1, jnp.int32)
            _, _, out = jax.lax.while_loop(cond, bod, (0, True, out0))
            first = jnp.maximum(out[:, 0:1], 0)
            out = jnp.where(out >= 0, out, first)
            o_ref[...] = out + base

        extract(s1_ref, k1, o1_ref)
        extract(s2_ref, k2, o2_ref)

    o1, o2 = pl.pallas_call(
        body,
        grid=(b, nb),
        in_specs=[
            pl.BlockSpec((1, 1, n), lambda i, j: (i, 0, 0)),
            pl.BlockSpec((1, 1, n), lambda i, j: (i, 0, 0)),
            pl.BlockSpec((1, 1, n), lambda i, j: (i, 0, 0)),
            pl.BlockSpec((m_blk, cq), lambda i, j: (i * nb + j, 0)),
        ],
        out_specs=[
            pl.BlockSpec((m_blk, k1), lambda i, j: (i * nb + j, 0)),
            pl.BlockSpec((m_blk, k2), lambda i, j: (i * nb + j, 0)),
        ],
        out_shape=[
            jax.ShapeDtypeStruct((b * m, k1), jnp.int32),
            jax.ShapeDtypeStruct((b * m, k2), jnp.int32),
        ],
        scratch_shapes=[
            pltpu.VMEM((m_blk, n), jnp.float32),
            pltpu.VMEM((m_blk, n), jnp.float32),
        ],
    )(px, py, pz, qtab)
    return o1, o2


def _sa_mlp(g, qtab, w1, w1q, b1, w2, b2, w3, b3, k, m_blk):
    rows = g.shape[0] // k
    cq = qtab.shape[1]
    cp = g.shape[1]
    c1 = w1.shape[1]
    c2 = w2.shape[1]
    c3 = w3.shape[1]
    nb = rows // m_blk

    def body(g_ref, q_ref, w1_ref, w1q_ref, b1_ref, w2_ref, b2_ref, w3_ref,
             b3_ref, o_ref):
        gv = g_ref[...]
        a = jnp.dot(gv, w1_ref[...], preferred_element_type=jnp.float32,
                    precision=jax.lax.Precision.HIGHEST)
        bq = jnp.dot(q_ref[...], w1q_ref[...], preferred_element_type=jnp.float32,
                    precision=jax.lax.Precision.HIGHEST)
        h = a.reshape(m_blk, k, c1) - bq[:, None, :] + b1_ref[...][None]
        h = jnp.maximum(h, 0.0).reshape(m_blk * k, c1)
        h = jnp.dot(h, w2_ref[...], preferred_element_type=jnp.float32,
                    precision=jax.lax.Precision.HIGHEST) + b2_ref[...]
        h = jnp.maximum(h, 0.0)
        h = jnp.dot(h, w3_ref[...], preferred_element_type=jnp.float32,
                    precision=jax.lax.Precision.HIGHEST) + b3_ref[...]
        h = jnp.maximum(h, 0.0)
        o_ref[...] = jnp.max(h.reshape(m_blk, k, c3), axis=1)

    return pl.pallas_call(
        body,
        grid=(nb,),
        in_specs=[
            pl.BlockSpec((m_blk * k, cp), lambda i: (i, 0)),
            pl.BlockSpec((m_blk, cq), lambda i: (i, 0)),
            pl.BlockSpec((cp, c1), lambda i: (0, 0)),
            pl.BlockSpec((cq, c1), lambda i: (0, 0)),
            pl.BlockSpec((1, c1), lambda i: (0, 0)),
            pl.BlockSpec((c1, c2), lambda i: (0, 0)),
            pl.BlockSpec((1, c2), lambda i: (0, 0)),
            pl.BlockSpec((c2, c3), lambda i: (0, 0)),
            pl.BlockSpec((1, c3), lambda i: (0, 0)),
        ],
        out_specs=pl.BlockSpec((m_blk, c3), lambda i: (i, 0)),
        out_shape=jax.ShapeDtypeStruct((rows, c3), jnp.float32),
    )(g, qtab, w1, w1q, b1, w2, b2, w3, b3)


def _fp_select(sx, sy, sz, ttab, s_pts, t_pts, t_blk):
    b = sx.shape[0]
    cq = ttab.shape[1]
    nb = t_pts // t_blk

    def body(sx_ref, sy_ref, sz_ref, t_ref, oi_ref, od_ref, d2_ref):
        bb = pl.program_id(0)
        base = bb * s_pts
        qx = t_ref[:, 0:1]
        qy = t_ref[:, 1:2]
        qz = t_ref[:, 2:3]
        qn = (qx * qx + qy * qy) + qz * qz
        x = sx_ref[0]
        y = sy_ref[0]
        z = sz_ref[0]
        pn = (x * x + y * y) + z * z
        p3 = jnp.concatenate([x, y, z], axis=0).astype(jnp.bfloat16)
        q3 = t_ref[:, 0:3].astype(jnp.bfloat16)
        ab = jnp.dot(q3, p3, preferred_element_type=jnp.float32)
        d2_ref[...] = (qn + pn) - 2.0 * ab
        iota = jax.lax.broadcasted_iota(jnp.int32, (t_blk, s_pts), 1).astype(jnp.float32)
        lane = jax.lax.broadcasted_iota(jnp.int32, (t_blk, 8), 1)
        oi = jnp.zeros((t_blk, 8), jnp.int32)
        od = jnp.zeros((t_blk, 8), jnp.float32)
        for s in range(3):
            mv = jnp.min(d2_ref[...], axis=1, keepdims=True)
            idxf = jnp.min(
                jnp.where(d2_ref[...] == mv, iota, _BIG), axis=1, keepdims=True
            )
            d2_ref[...] = jnp.where(iota == idxf, _BIG, d2_ref[...])
            oi = jnp.where(lane == s, idxf.astype(jnp.int32) + base, oi)
            od = jnp.where(lane == s, jnp.maximum(mv, 0.0), od)
        oi_ref[...] = oi
        od_ref[...] = od

    oi, od = pl.pallas_call(
        body,
        grid=(b, nb),
        in_specs=[
            pl.BlockSpec((1, 1, s_pts), lambda i, j: (i, 0, 0)),
            pl.BlockSpec((1, 1, s_pts), lambda i, j: (i, 0, 0)),
            pl.BlockSpec((1, 1, s_pts), lambda i, j: (i, 0, 0)),
            pl.BlockSpec((t_blk, cq), lambda i, j: (i * nb + j, 0)),
        ],
        out_specs=[
            pl.BlockSpec((t_blk, 8), lambda i, j: (i * nb + j, 0)),
            pl.BlockSpec((t_blk, 8), lambda i, j: (i * nb + j, 0)),
        ],
        out_shape=[
            jax.ShapeDtypeStruct((b * t_pts, 8), jnp.int32),
            jax.ShapeDtypeStruct((b * t_pts, 8), jnp.float32),
        ],
        scratch_shapes=[pltpu.VMEM((t_blk, s_pts), jnp.float32)],
    )(sx, sy, sz, ttab)
    return oi, od


def _fp_mlp(sf, dd, skip, w1a, w1b, b1, w2, b2, m_blk):
    rows = dd.shape[0]
    cs = sf.shape[1]
    csk = skip.shape[1]
    c1 = w1a.shape[1]
    c2 = w2.shape[1]
    nb = rows // m_blk

    def body(sf_ref, d_ref, sk_ref, w1a_ref, w1b_ref, b1_ref, w2_ref, b2_ref,
             o_ref):
        d = d_ref[...][:, 0:3]
        w = 1.0 / (d + 1e-8)
        ws = (w[:, 0:1] + w[:, 1:2]) + w[:, 2:3]
        w = w / ws
        sf3 = sf_ref[...].reshape(m_blk, 3, cs)
        interp = (sf3[:, 0, :] * w[:, 0:1] + sf3[:, 1, :] * w[:, 1:2]) + \
            sf3[:, 2, :] * w[:, 2:3]
        h = jnp.dot(sk_ref[...], w1a_ref[...], preferred_element_type=jnp.float32,
                    precision=jax.lax.Precision.HIGHEST)
        h = h + jnp.dot(interp, w1b_ref[...], preferred_element_type=jnp.float32,
                    precision=jax.lax.Precision.HIGHEST)
        h = jnp.maximum(h + b1_ref[...], 0.0)
        h = jnp.dot(h, w2_ref[...], preferred_element_type=jnp.float32,
                    precision=jax.lax.Precision.HIGHEST) + b2_ref[...]
        o_ref[...] = jnp.maximum(h, 0.0)

    return pl.pallas_call(
        body,
        grid=(nb,),
        in_specs=[
            pl.BlockSpec((m_blk * 3, cs), lambda i: (i, 0)),
            pl.BlockSpec((m_blk, 8), lambda i: (i, 0)),
            pl.BlockSpec((m_blk, csk), lambda i: (i, 0)),
            pl.BlockSpec((csk, c1), lambda i: (0, 0)),
            pl.BlockSpec((cs, c1), lambda i: (0, 0)),
            pl.BlockSpec((1, c1), lambda i: (0, 0)),
            pl.BlockSpec((c1, c2), lambda i: (0, 0)),
            pl.BlockSpec((1, c2), lambda i: (0, 0)),
        ],
        out_specs=pl.BlockSpec((m_blk, c2), lambda i: (i, 0)),
        out_shape=jax.ShapeDtypeStruct((rows, c2), jnp.float32),
    )(sf, dd, skip, w1a, w1b, b1, w2, b2)


def _prep_layer(layer):
    scale = layer["gamma"] / jnp.sqrt(1.0 + _BN_EPS)
    wt = (layer["w"] * scale[:, None]).T
    return wt, layer["beta"][None, :]


def _pad_rows(w, rows):
    return jnp.pad(w, ((0, rows - w.shape[0]), (0, 0)))


def _pad_cols(x, cols):
    return jnp.pad(x, ((0, 0), (0, cols - x.shape[1])))


def _pad16(c):
    return ((c + 15) // 16) * 16


def kernel(points, params):
    b, n0, _ = points.shape
    sizes = (n0,) + _NUM_POINTS

    tbl0 = _pad_cols(points.reshape(b * n0, 5), 16)
    tables = [tbl0]
    feat_tbls = [None]
    coords = []

    def col(tbl, j, npts):
        return tbl[:, j].reshape(b, 1, npts)

    coords.append((col(tbl0, 0, n0), col(tbl0, 1, n0), col(tbl0, 2, n0)))

    for i in range(4):
        n = sizes[i]
        m = sizes[i + 1]
        k1, k2 = _NSAMPLES[i]
        r1, r2 = _RADII[i]
        px, py, pz = coords[i]

        xs = px.reshape(b, n // 128, 128)
        ys = py.reshape(b, n // 128, 128)
        zs = pz.reshape(b, n // 128, 128)
        fps_idx = _fps(xs, ys, zs, m)

        qtab = _sc_gather(tables[i], fps_idx)

        m_blk = min(128, m)
        o1, o2 = _sa_select(px, py, pz, qtab, n, m, r1, r2, k1, k2, m_blk)

        g1 = _sc_gather(tables[i], o1.reshape(-1))
        g2 = _sc_gather(tables[i], o2.reshape(-1))

        outs = []
        for s, (g, kk) in enumerate(((g1, k1), (g2, k2))):
            mlp = params["sa"][i][s]
            w1t, b1 = _prep_layer(mlp[0])
            w2t, b2 = _prep_layer(mlp[1])
            w3t, b3 = _prep_layer(mlp[2])
            cp = tables[i].shape[1]
            cq = qtab.shape[1]
            w1 = _pad_rows(w1t, cp)
            w1q = _pad_rows(w1t[:3], cq)
            mb = max(8, min(128, 2048 // kk, m))
            outs.append(_sa_mlp(g, qtab, w1, w1q, b1, w2t, b2, w3t, b3, kk, mb))
        feat = jnp.concatenate(outs, axis=1)

        tables.append(_pad_cols(jnp.concatenate([qtab[:, :3], feat], axis=1),
                                _pad16(3 + feat.shape[1])))
        feat_tbls.append(feat)
        tq = tables[i + 1]
        coords.append((col(tq, 0, m), col(tq, 1, m), col(tq, 2, m)))

    skip0 = _pad_cols(points[..., 3:].reshape(b * n0, 2), 16)
    skips = [skip0, feat_tbls[1], feat_tbls[2], feat_tbls[3]]
    src_feats = feat_tbls[4]
    fp_outs = []
    for j in range(4):
        tgt = 3 - j
        src = 4 - j
        t_pts = sizes[tgt]
        s_pts = sizes[src]
        sx, sy, sz = coords[src]
        oi, od = _fp_select(sx, sy, sz, tables[tgt], s_pts, t_pts,
                            min(128, t_pts))
        idx3 = oi[:, :3].reshape(-1)
        sf = _sc_gather(src_feats, idx3)
        mlp = params["fp"][j]
        w1t, b1 = _prep_layer(mlp[0])
        w2t, b2 = _prep_layer(mlp[1])
        csk = skips[tgt].shape[1]
        cs = src_feats.shape[1]
        nsk = csk if tgt > 0 else 2
        w1a = _pad_rows(w1t[:nsk], csk)
        w1b = w1t[nsk:]
        assert w1b.shape[0] == cs, (w1b.shape, cs)
        out = _fp_mlp(sf, od, skips[tgt], w1a, w1b, b1, w2t, b2, 128)
        fp_outs.append(out)
        src_feats = out

    xyz_out = tuple(
        tables[i][:, :3].reshape(b, sizes[i], 3) for i in range(4)
    )
    feat_out = tuple(
        fp_outs[3 - i].reshape(b, sizes[i], -1).transpose(0, 2, 1)
        for i in range(4)
    )
    return xyz_out + feat_out
